# baseline (speedup 1.0000x reference)
import functools

import jax
import jax.numpy as jnp
from jax import lax
from jax.experimental import pallas as pl
from jax.experimental.pallas import tpu as pltpu
from jax.experimental.pallas import tpu_sc as plsc

N = 10000
E = 320000
D = 128
NC = 2
NS = 16
NW = NC * NS
LANES = 16
EPW = E // NW
CH = 80
NCHUNK = EPW // CH
RPT = N // NS
NPAD = 10112

_MESH = plsc.VectorSubcoreMesh(
    core_axis_name="c", subcore_axis_name="s", num_cores=NC, num_subcores=NS
)


@functools.partial(
    pl.kernel,
    out_type=jax.ShapeDtypeStruct((NW, N), jnp.float32),
    mesh=_MESH,
    scratch_types=[
        pltpu.VMEM((EPW,), jnp.int32),
        pltpu.VMEM((NPAD,), jnp.float32),
        pltpu.SemaphoreType.DMA,
    ],
    compiler_params=pltpu.CompilerParams(needs_layout_passes=False, use_tc_tiling_on_sc=False),
)
def _deg_kernel(ef_hbm, out_hbm, dstv, bins, sem):
    c = lax.axis_index("c")
    s = lax.axis_index("s")
    wid = c * NS + s

    base = pl.multiple_of(E + wid * EPW, 8)
    cp = pltpu.async_copy(ef_hbm.at[pl.ds(base, EPW)], dstv, sem)

    zeros = jnp.zeros((LANES,), jnp.float32)

    def _zero(i, carry):
        for k in range(8):
            bins[pl.ds((i * 8 + k) * LANES, LANES)] = zeros
        return carry

    lax.fori_loop(0, NPAD // (8 * LANES), _zero, 0)
    cp.wait()

    ones = jnp.ones((LANES,), jnp.float32)

    def _hist(i, carry):
        for k in range(5):
            idx = dstv[pl.ds((i * 5 + k) * LANES, LANES)]
            plsc.addupdate_scatter(bins, [idx], ones)
        return carry

    lax.fori_loop(0, EPW // (5 * LANES), _hist, 0)
    pltpu.sync_copy(bins.at[pl.ds(0, N)], out_hbm.at[wid])


def _scale_body(degp_ref, x_ref, dinv_ref, xs_ref, xsh_ref):
    deg = jnp.sum(degp_ref[...], axis=0) + 1.0
    dinv = lax.rsqrt(deg)[:, None]
    dinv_ref[...] = dinv
    xs = x_ref[...] * dinv
    xs_ref[...] = xs
    xsh_ref[...] = xs.astype(jnp.bfloat16)


_TCBLK = 2560


def _scale_call(degp, x):
    return pl.pallas_call(
        _scale_body,
        grid=((N + _TCBLK - 1) // _TCBLK,),
        in_specs=[
            pl.BlockSpec((NW, _TCBLK), lambda i: (0, i)),
            pl.BlockSpec((_TCBLK, D), lambda i: (i, 0)),
        ],
        out_specs=(
            pl.BlockSpec((_TCBLK, 1), lambda i: (i, 0)),
            pl.BlockSpec((_TCBLK, D), lambda i: (i, 0)),
            pl.BlockSpec((_TCBLK, D), lambda i: (i, 0)),
        ),
        out_shape=(
            jax.ShapeDtypeStruct((N, 1), jnp.float32),
            jax.ShapeDtypeStruct((N, D), jnp.float32),
            jax.ShapeDtypeStruct((N, D), jnp.bfloat16),
        ),
    )(degp, x)


NB = 2
assert NCHUNK % 2 == 1 and NCHUNK >= 5
ZR = 7 * CH
ZTAIL = RPT - ZR


@functools.partial(
    pl.kernel,
    out_type=jax.ShapeDtypeStruct((NC, N, D), jnp.float32),
    mesh=_MESH,
    scratch_types=[
        pltpu.VMEM((EPW,), jnp.int32),
        pltpu.VMEM((EPW,), jnp.int32),
        pltpu.VMEM((NB, CH, D), jnp.bfloat16),
        pltpu.VMEM((NB, CH, D), jnp.float32),
        pltpu.VMEM_SHARED((N, D), jnp.float32),
        pltpu.SemaphoreType.DMA,
        pltpu.SemaphoreType.DMA,
        pltpu.SemaphoreType.DMA,
        pltpu.SemaphoreType.DMA,
    ],
    compiler_params=pltpu.CompilerParams(needs_layout_passes=False, use_tc_tiling_on_sc=False),
)
def _scatter_kernel(ef_hbm, xsh_hbm, out_hbm,
                    src_all, dst_all, bro, fro, acc_sh,
                    sg0, sg1, ss0, ss1):
    c = lax.axis_index("c")
    s = lax.axis_index("s")
    wid = c * NS + s
    sg = (sg0, sg1)
    ss = (ss0, ss1)

    ebase = pl.multiple_of(wid * EPW, 8)
    c0 = pltpu.async_copy(ef_hbm.at[pl.ds(ebase, EPW)], src_all, ss0)
    c1 = pltpu.async_copy(ef_hbm.at[pl.ds(E + ebase, EPW)], dst_all, ss1)
    c0.wait()
    c1.wait()

    def _gather(i, b):
        pltpu.async_copy(
            xsh_hbm.at[src_all.at[pl.ds(i * CH, CH)]], bro.at[b], sg[b])

    def _wait_gather(i, b):
        pltpu.make_async_copy(
            xsh_hbm.at[src_all.at[pl.ds(i * CH, CH)]], bro.at[b], sg[b]).wait()

    def _scatter_start(i, b):
        pltpu.async_copy(
            fro.at[b], acc_sh.at[dst_all.at[pl.ds(i * CH, CH)]], ss[b],
            add=True)

    def _wait_scatter(i, b):
        pltpu.make_async_copy(
            fro.at[b], acc_sh.at[dst_all.at[pl.ds(i * CH, CH)]], ss[b]).wait()

    MASK = jnp.full((LANES,), -65536, jnp.int32)

    def _convert(b):
        def _crow(r, carry):
            for q in range(D // 32):
                v = plsc.bitcast(bro[b, r, pl.ds(q * 32, 32)], jnp.int32)
                lo = lax.shift_left(v, 16)
                hi = lax.bitwise_and(v, MASK)
                fro[b, r, pl.ds(q * 32, LANES)] = plsc.bitcast(lo, jnp.float32)
                fro[b, r, pl.ds(q * 32 + LANES, LANES)] = plsc.bitcast(
                    hi, jnp.float32)
            return carry

        lax.fori_loop(0, CH, _crow, 0)

    _gather(0, 0)
    _gather(1, 1)

    zflat = jnp.zeros((LANES,), jnp.float32)
    zbuf = fro.at[0]

    def _zrow(r, carry):
        for k in range(D // LANES):
            zbuf[r, pl.ds(k * LANES, LANES)] = zflat
        return carry

    lax.fori_loop(0, CH, _zrow, 0)
    rbase = s * RPT
    for k in range(7):
        pltpu.sync_copy(zbuf, acc_sh.at[pl.ds(rbase + k * CH, CH)])
    pltpu.sync_copy(zbuf.at[pl.ds(0, ZTAIL)], acc_sh.at[pl.ds(rbase + ZR, ZTAIL)])
    plsc.subcore_barrier()

    for i in (0, 1):
        _wait_gather(i, i)
        _convert(i)
        _scatter_start(i, i)
        _gather(i + 2, i)

    def _pair(j, carry):
        i = j * 2
        for b in range(2):
            _wait_gather(i + b, b)
            _wait_scatter(i + b - 2, b)
            _convert(b)
            _scatter_start(i + b, b)
            _gather(i + 2 + b, b)
        return carry

    lax.fori_loop(1, (NCHUNK - 5) // 2 + 1, _pair, 0)

    t = NCHUNK - 3
    _wait_gather(t, 0)
    _wait_scatter(t - 2, 0)
    _convert(0)
    _scatter_start(t, 0)
    _gather(t + 2, 0)
    _wait_gather(t + 1, 1)
    _wait_scatter(t - 1, 1)
    _convert(1)
    _scatter_start(t + 1, 1)
    _wait_gather(t + 2, 0)
    _wait_scatter(t, 0)
    _convert(0)
    _scatter_start(t + 2, 0)
    _wait_scatter(t + 1, 1)
    _wait_scatter(t + 2, 0)

    plsc.subcore_barrier()
    pltpu.sync_copy(acc_sh.at[pl.ds(rbase, RPT)], out_hbm.at[c].at[pl.ds(rbase, RPT)])


def _out_body(accp_ref, xs_ref, dinv_ref, wg_ref, w_ref, b_ref, out_ref):
    dinv = dinv_ref[...]
    yg = (accp_ref[0] + accp_ref[1]) * dinv
    ys = xs_ref[...] * dinv
    out_ref[...] = (
        jnp.dot(yg, wg_ref[...], preferred_element_type=jnp.float32)
        + jnp.dot(ys, w_ref[...], preferred_element_type=jnp.float32)
        + b_ref[...]
    )


def _out_call(accp, xs, dinv, Wg, W, b2):
    return pl.pallas_call(
        _out_body,
        grid=((N + _TCBLK - 1) // _TCBLK,),
        in_specs=[
            pl.BlockSpec((NC, _TCBLK, D), lambda i: (0, i, 0)),
            pl.BlockSpec((_TCBLK, D), lambda i: (i, 0)),
            pl.BlockSpec((_TCBLK, 1), lambda i: (i, 0)),
            pl.BlockSpec((D, D), lambda i: (0, 0)),
            pl.BlockSpec((D, D), lambda i: (0, 0)),
            pl.BlockSpec((1, D), lambda i: (0, 0)),
        ],
        out_specs=pl.BlockSpec((_TCBLK, D), lambda i: (i, 0)),
        out_shape=jax.ShapeDtypeStruct((N, D), jnp.float32),
    )(accp, xs, dinv, Wg, W, b2)


_GPERM = [
    32 * q + (2 * m if m < 16 else 2 * (m - 16) + 1)
    for q in range(D // 32)
    for m in range(32)
]


def kernel(x, edge_index, W, b):
    ef = edge_index.reshape(2 * E)
    degp = _deg_kernel(ef)
    dinv, xs, xsh = _scale_call(degp, x)
    accp = _scatter_kernel(ef, xsh)
    Wg = W[jnp.array(_GPERM), :]
    return _out_call(accp, xs, dinv, Wg, W, b.reshape(1, D))


# speedup vs baseline: 1.8250x; 1.8250x over previous
import functools

import jax
import jax.numpy as jnp
from jax import lax
from jax.experimental import pallas as pl
from jax.experimental.pallas import tpu as pltpu
from jax.experimental.pallas import tpu_sc as plsc

N = 10000
E = 320000
D = 128
NC = 2
NS = 16
NW = NC * NS
LANES = 16
EPW = E // NW
CH = 80
NCHUNK = EPW // CH
RPT = N // NS
NPAD = 10112

_MESH = plsc.VectorSubcoreMesh(
    core_axis_name="c", subcore_axis_name="s", num_cores=NC, num_subcores=NS
)


@functools.partial(
    pl.kernel,
    out_type=jax.ShapeDtypeStruct((NW, N), jnp.float32),
    mesh=_MESH,
    scratch_types=[
        pltpu.VMEM((EPW,), jnp.int32),
        pltpu.VMEM((NPAD,), jnp.float32),
        pltpu.SemaphoreType.DMA,
    ],
    compiler_params=pltpu.CompilerParams(needs_layout_passes=False, use_tc_tiling_on_sc=False),
)
def _deg_kernel(ef_hbm, out_hbm, dstv, bins, sem):
    c = lax.axis_index("c")
    s = lax.axis_index("s")
    wid = c * NS + s

    base = pl.multiple_of(E + wid * EPW, 8)
    cp = pltpu.async_copy(ef_hbm.at[pl.ds(base, EPW)], dstv, sem)

    zeros = jnp.zeros((LANES,), jnp.float32)

    def _zero(i, carry):
        for k in range(8):
            bins[pl.ds((i * 8 + k) * LANES, LANES)] = zeros
        return carry

    lax.fori_loop(0, NPAD // (8 * LANES), _zero, 0)
    cp.wait()

    ones = jnp.ones((LANES,), jnp.float32)

    def _hist(i, carry):
        for k in range(5):
            idx = dstv[pl.ds((i * 5 + k) * LANES, LANES)]
            plsc.addupdate_scatter(bins, [idx], ones)
        return carry

    lax.fori_loop(0, EPW // (5 * LANES), _hist, 0)
    pltpu.sync_copy(bins.at[pl.ds(0, N)], out_hbm.at[wid])


def _scale_body(degp_ref, x_ref, dinv_ref, xs_ref):
    deg = jnp.sum(degp_ref[...], axis=0) + 1.0
    dinv = lax.rsqrt(deg)[:, None]
    dinv_ref[...] = dinv
    xs_ref[...] = x_ref[...] * dinv


_TCBLK = 2560


def _scale_call(degp, x):
    return pl.pallas_call(
        _scale_body,
        grid=((N + _TCBLK - 1) // _TCBLK,),
        in_specs=[
            pl.BlockSpec((NW, _TCBLK), lambda i: (0, i)),
            pl.BlockSpec((_TCBLK, D), lambda i: (i, 0)),
        ],
        out_specs=(
            pl.BlockSpec((_TCBLK, 1), lambda i: (i, 0)),
            pl.BlockSpec((_TCBLK, D), lambda i: (i, 0)),
        ),
        out_shape=(
            jax.ShapeDtypeStruct((N, 1), jnp.float32),
            jax.ShapeDtypeStruct((N, D), jnp.float32),
        ),
    )(degp, x)


NB = 3
assert (NCHUNK - 5) % NB == 0 and NCHUNK >= 8
ZR = 7 * CH
ZTAIL = RPT - ZR


@functools.partial(
    pl.kernel,
    out_type=jax.ShapeDtypeStruct((NC, N, D), jnp.float32),
    mesh=_MESH,
    scratch_types=[
        pltpu.VMEM((EPW,), jnp.int32),
        pltpu.VMEM((EPW,), jnp.int32),
        pltpu.VMEM((NB, CH, D), jnp.float32),
        pltpu.VMEM_SHARED((N, D), jnp.float32),
        pltpu.SemaphoreType.DMA,
        pltpu.SemaphoreType.DMA,
        pltpu.SemaphoreType.DMA,
        pltpu.SemaphoreType.DMA,
        pltpu.SemaphoreType.DMA,
        pltpu.SemaphoreType.DMA,
    ],
    compiler_params=pltpu.CompilerParams(needs_layout_passes=False, use_tc_tiling_on_sc=False),
)
def _scatter_kernel(ef_hbm, xs_hbm, out_hbm,
                    src_all, dst_all, rows, acc_sh,
                    sg0, sg1, sg2, ss0, ss1, ss2):
    c = lax.axis_index("c")
    s = lax.axis_index("s")
    wid = c * NS + s
    sg = (sg0, sg1, sg2)
    ss = (ss0, ss1, ss2)

    ebase = pl.multiple_of(wid * EPW, 8)
    c0 = pltpu.async_copy(ef_hbm.at[pl.ds(ebase, EPW)], src_all, ss0)
    c1 = pltpu.async_copy(ef_hbm.at[pl.ds(E + ebase, EPW)], dst_all, ss1)
    c0.wait()
    c1.wait()

    def _gather(i, b):
        pltpu.async_copy(
            xs_hbm.at[src_all.at[pl.ds(i * CH, CH)]], rows.at[b], sg[b])

    def _wait_gather(i, b):
        pltpu.make_async_copy(
            xs_hbm.at[src_all.at[pl.ds(i * CH, CH)]], rows.at[b], sg[b]).wait()

    def _scatter(i, b):
        pltpu.async_copy(
            rows.at[b], acc_sh.at[dst_all.at[pl.ds(i * CH, CH)]], ss[b],
            add=True).wait()

    _gather(1, 1)
    _gather(2, 2)

    zflat = jnp.zeros((LANES,), jnp.float32)
    zbuf = rows.at[0]

    def _zrow(r, carry):
        for k in range(D // LANES):
            zbuf[r, pl.ds(k * LANES, LANES)] = zflat
        return carry

    lax.fori_loop(0, CH, _zrow, 0)
    rbase = s * RPT
    for k in range(7):
        pltpu.sync_copy(zbuf, acc_sh.at[pl.ds(rbase + k * CH, CH)])
    pltpu.sync_copy(zbuf.at[pl.ds(0, ZTAIL)], acc_sh.at[pl.ds(rbase + ZR, ZTAIL)])
    _gather(0, 0)
    plsc.subcore_barrier()

    def _trio(j, carry):
        i = j * NB
        for b in range(NB):
            _wait_gather(i + b, b)
            _scatter(i + b, b)
            _gather(i + NB + b, b)
        return carry

    lax.fori_loop(0, (NCHUNK - 5) // NB, _trio, 0)

    t = NCHUNK - 5
    _wait_gather(t, 0)
    _scatter(t, 0)
    _gather(t + 3, 0)
    _wait_gather(t + 1, 1)
    _scatter(t + 1, 1)
    _gather(t + 4, 1)
    _wait_gather(t + 2, 2)
    _scatter(t + 2, 2)
    _wait_gather(t + 3, 0)
    _scatter(t + 3, 0)
    _wait_gather(t + 4, 1)
    _scatter(t + 4, 1)

    plsc.subcore_barrier()
    pltpu.sync_copy(acc_sh.at[pl.ds(rbase, RPT)], out_hbm.at[c].at[pl.ds(rbase, RPT)])


def _out_body(accp_ref, xs_ref, dinv_ref, w_ref, b_ref, out_ref):
    y = (accp_ref[0] + accp_ref[1] + xs_ref[...]) * dinv_ref[...]
    out_ref[...] = (
        jnp.dot(y, w_ref[...], preferred_element_type=jnp.float32) + b_ref[...]
    )


def _out_call(accp, xs, dinv, W, b2):
    return pl.pallas_call(
        _out_body,
        grid=((N + _TCBLK - 1) // _TCBLK,),
        in_specs=[
            pl.BlockSpec((NC, _TCBLK, D), lambda i: (0, i, 0)),
            pl.BlockSpec((_TCBLK, D), lambda i: (i, 0)),
            pl.BlockSpec((_TCBLK, 1), lambda i: (i, 0)),
            pl.BlockSpec((D, D), lambda i: (0, 0)),
            pl.BlockSpec((1, D), lambda i: (0, 0)),
        ],
        out_specs=pl.BlockSpec((_TCBLK, D), lambda i: (i, 0)),
        out_shape=jax.ShapeDtypeStruct((N, D), jnp.float32),
    )(accp, xs, dinv, W, b2)


def kernel(x, edge_index, W, b):
    ef = edge_index.reshape(2 * E)
    degp = _deg_kernel(ef)
    dinv, xs = _scale_call(degp, x)
    accp = _scatter_kernel(ef, xs)
    return _out_call(accp, xs, dinv, W, b.reshape(1, D))
